# baseline (speedup 1.0000x reference)
import functools

import numpy as np
import jax
import jax.numpy as jnp
from jax import lax
from jax.experimental import pallas as pl
from jax.experimental.pallas import tpu as pltpu
from jax.experimental.pallas import tpu_sc as plsc

N = 10000
C = 128
E = 320000
NC = 2
NS = 16
NW = NC * NS
NP = 10240
ROWS_PER_TILE = NP // NS
CW = 128
CH = 80
NPASS = 2
CPP = CH // NPASS
BR2 = 1000
EPT = CH * CW
EP = NW * EPT
DEGW = 16
BR = 640

_mesh = plsc.VectorSubcoreMesh(core_axis_name="c", subcore_axis_name="s")


@functools.partial(
    pl.kernel,
    out_type=jax.ShapeDtypeStruct((NC, NP, DEGW), jnp.float32),
    mesh=_mesh,
    scratch_types=[
        pltpu.VMEM((CH, CW), jnp.int32),
        pltpu.VMEM((CW, DEGW), jnp.float32),
        pltpu.VMEM((CW, DEGW), jnp.float32),
        pltpu.VMEM_SHARED((NP, DEGW), jnp.float32),
    ],
)
def _deg_kernel(ei_hbm, deg_out, dst_v, ones_a, zbuf, deg_sh):
    c = lax.axis_index("c")
    s = lax.axis_index("s")
    pltpu.sync_copy(ei_hbm.at[1, c, s], dst_v)

    def fill(i, carry):
        ones_a[i, :] = jnp.ones((DEGW,), jnp.float32)
        zbuf[i, :] = jnp.zeros((DEGW,), jnp.float32)
        return carry

    lax.fori_loop(0, CW, fill, 0)
    base = s * ROWS_PER_TILE
    for k in range(ROWS_PER_TILE // CW):
        pltpu.sync_copy(zbuf, deg_sh.at[pl.ds(base + k * CW, CW)])
    plsc.subcore_barrier()

    def body(j, carry):
        pltpu.sync_copy(ones_a, deg_sh.at[dst_v.at[j]], add=True)
        return carry

    lax.fori_loop(0, CH, body, 0)
    plsc.subcore_barrier()
    for k in range(ROWS_PER_TILE // CW):
        pltpu.sync_copy(deg_sh.at[pl.ds(base + k * CW, CW)], zbuf)
        pltpu.sync_copy(zbuf, deg_out.at[c, pl.ds(base + k * CW, CW)])


@functools.partial(
    pl.kernel,
    out_type=jax.ShapeDtypeStruct((NC, NP, C), jnp.float32),
    mesh=_mesh,
    scratch_types=[
        pltpu.VMEM((CPP, CW), jnp.int32),
        pltpu.VMEM((CPP, CW), jnp.int32),
        pltpu.VMEM((CW, C), jnp.float32),
        pltpu.VMEM((CW, C), jnp.float32),
        pltpu.VMEM_SHARED((NP, C), jnp.float32),
        pltpu.SemaphoreType.DMA,
        pltpu.SemaphoreType.DMA,
        pltpu.SemaphoreType.DMA,
        pltpu.SemaphoreType.DMA,
    ],
)
def _agg_kernel(ei_hbm, hp_hbm, acc_out,
                src_v, dst_v, buf0, buf1, acc_sh, g0, g1, s0, s1):
    c = lax.axis_index("c")
    s = lax.axis_index("s")

    def zf(i, carry):
        for k in range(C // 16):
            buf0[i, pl.ds(k * 16, 16)] = jnp.zeros((16,), jnp.float32)
        return carry

    lax.fori_loop(0, CW, zf, 0)
    base = s * ROWS_PER_TILE
    for k in range(ROWS_PER_TILE // CW):
        pltpu.sync_copy(buf0, acc_sh.at[pl.ds(base + k * CW, CW)])
    plsc.subcore_barrier()

    bufs = (buf0, buf1)
    gsems = (g0, g1)
    ssems = (s0, s1)

    def body(jj, carry):
        for bi in range(2):
            j = 2 * jj + bi
            buf, gs, ss = bufs[bi], gsems[bi], ssems[bi]
            pltpu.make_async_copy(hp_hbm.at[src_v.at[j]], buf, gs).wait()
            pltpu.async_copy(buf, acc_sh.at[dst_v.at[j]], ss, add=True).wait()
            jn = jnp.minimum(j + 2, CPP - 1)
            pltpu.async_copy(hp_hbm.at[src_v.at[jn]], buf, gs)
        return carry

    for p in range(NPASS):
        pltpu.sync_copy(ei_hbm.at[0, c, s, pl.ds(p * CPP, CPP)], src_v)
        pltpu.sync_copy(ei_hbm.at[1, c, s, pl.ds(p * CPP, CPP)], dst_v)
        pltpu.async_copy(hp_hbm.at[src_v.at[0]], buf0, g0)
        pltpu.async_copy(hp_hbm.at[src_v.at[1]], buf1, g1)
        lax.fori_loop(0, CPP // 2, body, 0)
        pltpu.make_async_copy(hp_hbm.at[src_v.at[CPP - 1]], buf0, g0).wait()
        pltpu.make_async_copy(hp_hbm.at[src_v.at[CPP - 1]], buf1, g1).wait()
    plsc.subcore_barrier()
    for k in range(ROWS_PER_TILE // CW):
        pltpu.sync_copy(acc_sh.at[pl.ds(base + k * CW, CW)], buf0)
        pltpu.sync_copy(buf0, acc_out.at[c, pl.ds(base + k * CW, CW)])


def _matmul_body(x_ref, w_ref, h_ref):
    h_ref[...] = jnp.dot(x_ref[...], w_ref[...],
                         preferred_element_type=jnp.float32)


_matmul = pl.pallas_call(
    _matmul_body,
    grid=(N // BR2,),
    in_specs=[
        pl.BlockSpec((BR2, C), lambda i: (i, 0)),
        pl.BlockSpec((C, C), lambda i: (0, 0)),
    ],
    out_specs=pl.BlockSpec((BR2, C), lambda i: (i, 0)),
    out_shape=jax.ShapeDtypeStruct((NP, C), jnp.float32),
)


def _scale_body(h_ref, deg_ref, hp_ref, dis_ref):
    dtot = deg_ref[0] + deg_ref[1] + 1.0
    dis = lax.rsqrt(dtot)[:, 0:1]
    hp_ref[...] = h_ref[...] * dis
    dis_ref[...] = dis


_scale = pl.pallas_call(
    _scale_body,
    grid=(NP // 1024,),
    in_specs=[
        pl.BlockSpec((1024, C), lambda i: (i, 0)),
        pl.BlockSpec((NC, 1024, DEGW), lambda i: (0, i, 0)),
    ],
    out_specs=[
        pl.BlockSpec((1024, C), lambda i: (i, 0)),
        pl.BlockSpec((1024, 1), lambda i: (i, 0)),
    ],
    out_shape=[
        jax.ShapeDtypeStruct((NP, C), jnp.float32),
        jax.ShapeDtypeStruct((NP, 1), jnp.float32),
    ],
)


def _finish_body(a_ref, hp_ref, dis_ref, b_ref, pw_ref, o_ref):
    t = (a_ref[0] + a_ref[1] + hp_ref[...]) * dis_ref[...] + b_ref[...]
    o_ref[...] = jnp.where(t >= 0.0, t, pw_ref[...] * t)


_finish = pl.pallas_call(
    _finish_body,
    grid=(N // BR2,),
    in_specs=[
        pl.BlockSpec((NC, BR2, C), lambda i: (0, i, 0)),
        pl.BlockSpec((BR2, C), lambda i: (i, 0)),
        pl.BlockSpec((BR2, 1), lambda i: (i, 0)),
        pl.BlockSpec((1, C), lambda i: (0, 0)),
        pl.BlockSpec((1, C), lambda i: (0, 0)),
    ],
    out_specs=pl.BlockSpec((BR2, C), lambda i: (i, 0)),
    out_shape=jax.ShapeDtypeStruct((N, C), jnp.float32),
)


_PAD2D = (np.arange(EP - E, dtype=np.int32) % (NP - N) + N).reshape(-1, CW)


def kernel(x, edge_index, W, b, prelu_w):
    ei = edge_index.astype(jnp.int32).reshape(2, E // CW, CW)
    pad3 = jnp.broadcast_to(_PAD2D, (2,) + _PAD2D.shape)
    eip = jnp.concatenate([ei, pad3], axis=1).reshape(2, NC, NS, CH, CW)

    deg = _deg_kernel(eip)
    h = _matmul(x, W)
    hp, dis = _scale(h, deg)
    acc = _agg_kernel(eip, hp)
    return _finish(acc, hp, dis, b.reshape(1, C), prelu_w.reshape(1, C))


# speedup vs baseline: 1.0002x; 1.0002x over previous
import functools

import numpy as np
import jax
import jax.numpy as jnp
from jax import lax
from jax.experimental import pallas as pl
from jax.experimental.pallas import tpu as pltpu
from jax.experimental.pallas import tpu_sc as plsc

N = 10000
C = 128
E = 320000
NC = 2
NS = 16
NW = NC * NS
NP = 10240
ROWS_PER_TILE = NP // NS
CW = 128
CH = 80
NPASS = 2
CPP = CH // NPASS
BR2 = 1000
EPT = CH * CW
EP = NW * EPT
DEGW = 16
BR = 640

_mesh = plsc.VectorSubcoreMesh(core_axis_name="c", subcore_axis_name="s")


@functools.partial(
    pl.kernel,
    out_type=jax.ShapeDtypeStruct((NC, NP, DEGW), jnp.float32),
    mesh=_mesh,
    scratch_types=[
        pltpu.VMEM((CH, CW), jnp.int32),
        pltpu.VMEM((CW, DEGW), jnp.float32),
        pltpu.VMEM((CW, DEGW), jnp.float32),
        pltpu.VMEM_SHARED((NP, DEGW), jnp.float32),
    ],
)
def _deg_kernel(ei_hbm, deg_out, dst_v, ones_a, zbuf, deg_sh):
    c = lax.axis_index("c")
    s = lax.axis_index("s")
    pltpu.sync_copy(ei_hbm.at[1, c, s], dst_v)

    def fill(i, carry):
        ones_a[i, :] = jnp.ones((DEGW,), jnp.float32)
        zbuf[i, :] = jnp.zeros((DEGW,), jnp.float32)
        return carry

    lax.fori_loop(0, CW, fill, 0)
    base = s * ROWS_PER_TILE
    for k in range(ROWS_PER_TILE // CW):
        pltpu.sync_copy(zbuf, deg_sh.at[pl.ds(base + k * CW, CW)])
    plsc.subcore_barrier()

    def body(j, carry):
        pltpu.sync_copy(ones_a, deg_sh.at[dst_v.at[j]], add=True)
        return carry

    lax.fori_loop(0, CH, body, 0)
    plsc.subcore_barrier()
    pltpu.sync_copy(deg_sh.at[pl.ds(base, ROWS_PER_TILE)],
                    deg_out.at[c, pl.ds(base, ROWS_PER_TILE)])


@functools.partial(
    pl.kernel,
    out_type=jax.ShapeDtypeStruct((NC, NP, C), jnp.float32),
    mesh=_mesh,
    scratch_types=[
        pltpu.VMEM((CPP, CW), jnp.int32),
        pltpu.VMEM((CPP, CW), jnp.int32),
        pltpu.VMEM((CW, C), jnp.float32),
        pltpu.VMEM((CW, C), jnp.float32),
        pltpu.VMEM_SHARED((NP, C), jnp.float32),
        pltpu.SemaphoreType.DMA,
        pltpu.SemaphoreType.DMA,
        pltpu.SemaphoreType.DMA,
        pltpu.SemaphoreType.DMA,
    ],
)
def _agg_kernel(ei_hbm, hp_hbm, acc_out,
                src_v, dst_v, buf0, buf1, acc_sh, g0, g1, s0, s1):
    c = lax.axis_index("c")
    s = lax.axis_index("s")

    def zf(i, carry):
        for k in range(C // 16):
            buf0[i, pl.ds(k * 16, 16)] = jnp.zeros((16,), jnp.float32)
        return carry

    lax.fori_loop(0, CW, zf, 0)
    base = s * ROWS_PER_TILE
    for k in range(ROWS_PER_TILE // CW):
        pltpu.sync_copy(buf0, acc_sh.at[pl.ds(base + k * CW, CW)])
    plsc.subcore_barrier()

    bufs = (buf0, buf1)
    gsems = (g0, g1)
    ssems = (s0, s1)

    def body(jj, carry):
        for bi in range(2):
            j = 2 * jj + bi
            buf, gs, ss = bufs[bi], gsems[bi], ssems[bi]
            pltpu.make_async_copy(hp_hbm.at[src_v.at[j]], buf, gs).wait()
            pltpu.async_copy(buf, acc_sh.at[dst_v.at[j]], ss, add=True).wait()
            jn = jnp.minimum(j + 2, CPP - 1)
            pltpu.async_copy(hp_hbm.at[src_v.at[jn]], buf, gs)
        return carry

    for p in range(NPASS):
        pltpu.sync_copy(ei_hbm.at[0, c, s, pl.ds(p * CPP, CPP)], src_v)
        pltpu.sync_copy(ei_hbm.at[1, c, s, pl.ds(p * CPP, CPP)], dst_v)
        pltpu.async_copy(hp_hbm.at[src_v.at[0]], buf0, g0)
        pltpu.async_copy(hp_hbm.at[src_v.at[1]], buf1, g1)
        lax.fori_loop(0, CPP // 2, body, 0)
        pltpu.make_async_copy(hp_hbm.at[src_v.at[CPP - 1]], buf0, g0).wait()
        pltpu.make_async_copy(hp_hbm.at[src_v.at[CPP - 1]], buf1, g1).wait()
    plsc.subcore_barrier()
    pltpu.sync_copy(acc_sh.at[pl.ds(base, ROWS_PER_TILE)],
                    acc_out.at[c, pl.ds(base, ROWS_PER_TILE)])


def _matmul_body(x_ref, w_ref, h_ref):
    h_ref[...] = jnp.dot(x_ref[...], w_ref[...],
                         preferred_element_type=jnp.float32)


_matmul = pl.pallas_call(
    _matmul_body,
    grid=(N // BR2,),
    in_specs=[
        pl.BlockSpec((BR2, C), lambda i: (i, 0)),
        pl.BlockSpec((C, C), lambda i: (0, 0)),
    ],
    out_specs=pl.BlockSpec((BR2, C), lambda i: (i, 0)),
    out_shape=jax.ShapeDtypeStruct((NP, C), jnp.float32),
)


def _scale_body(h_ref, deg_ref, hp_ref, dis_ref):
    dtot = deg_ref[0] + deg_ref[1] + 1.0
    dis = lax.rsqrt(dtot)[:, 0:1]
    hp_ref[...] = h_ref[...] * dis
    dis_ref[...] = dis


_scale = pl.pallas_call(
    _scale_body,
    grid=(NP // 1024,),
    in_specs=[
        pl.BlockSpec((1024, C), lambda i: (i, 0)),
        pl.BlockSpec((NC, 1024, DEGW), lambda i: (0, i, 0)),
    ],
    out_specs=[
        pl.BlockSpec((1024, C), lambda i: (i, 0)),
        pl.BlockSpec((1024, 1), lambda i: (i, 0)),
    ],
    out_shape=[
        jax.ShapeDtypeStruct((NP, C), jnp.float32),
        jax.ShapeDtypeStruct((NP, 1), jnp.float32),
    ],
)


def _finish_body(a_ref, hp_ref, dis_ref, b_ref, pw_ref, o_ref):
    t = (a_ref[0] + a_ref[1] + hp_ref[...]) * dis_ref[...] + b_ref[...]
    o_ref[...] = jnp.where(t >= 0.0, t, pw_ref[...] * t)


_finish = pl.pallas_call(
    _finish_body,
    grid=(N // BR2,),
    in_specs=[
        pl.BlockSpec((NC, BR2, C), lambda i: (0, i, 0)),
        pl.BlockSpec((BR2, C), lambda i: (i, 0)),
        pl.BlockSpec((BR2, 1), lambda i: (i, 0)),
        pl.BlockSpec((1, C), lambda i: (0, 0)),
        pl.BlockSpec((1, C), lambda i: (0, 0)),
    ],
    out_specs=pl.BlockSpec((BR2, C), lambda i: (i, 0)),
    out_shape=jax.ShapeDtypeStruct((N, C), jnp.float32),
)


_PAD2D = (np.arange(EP - E, dtype=np.int32) % (NP - N) + N).reshape(-1, CW)


def kernel(x, edge_index, W, b, prelu_w):
    ei = edge_index.astype(jnp.int32).reshape(2, E // CW, CW)
    pad3 = jnp.broadcast_to(_PAD2D, (2,) + _PAD2D.shape)
    eip = jnp.concatenate([ei, pad3], axis=1).reshape(2, NC, NS, CH, CW)

    deg = _deg_kernel(eip)
    h = _matmul(x, W)
    hp, dis = _scale(h, deg)
    acc = _agg_kernel(eip, hp)
    return _finish(acc, hp, dis, b.reshape(1, C), prelu_w.reshape(1, C))


# speedup vs baseline: 1.0124x; 1.0122x over previous
import functools

import numpy as np
import jax
import jax.numpy as jnp
from jax import lax
from jax.experimental import pallas as pl
from jax.experimental.pallas import tpu as pltpu
from jax.experimental.pallas import tpu_sc as plsc

N = 10000
C = 128
E = 320000
NC = 2
NS = 16
NW = NC * NS
NP = 10240
ROWS_PER_TILE = NP // NS
CW = 128
CH = 80
NPASS = 2
CPP = CH // NPASS
BR2 = 2000
EPT = CH * CW
EP = NW * EPT
DEGW = 16
BR = 640

_mesh = plsc.VectorSubcoreMesh(core_axis_name="c", subcore_axis_name="s")


@functools.partial(
    pl.kernel,
    out_type=jax.ShapeDtypeStruct((NC, NP, DEGW), jnp.float32),
    mesh=_mesh,
    scratch_types=[
        pltpu.VMEM((CH, CW), jnp.int32),
        pltpu.VMEM((CW, DEGW), jnp.float32),
        pltpu.VMEM((CW, DEGW), jnp.float32),
        pltpu.VMEM_SHARED((NP, DEGW), jnp.float32),
    ],
)
def _deg_kernel(ei_hbm, deg_out, dst_v, ones_a, zbuf, deg_sh):
    c = lax.axis_index("c")
    s = lax.axis_index("s")
    pltpu.sync_copy(ei_hbm.at[1, c, s], dst_v)

    def fill(i, carry):
        ones_a[i, :] = jnp.ones((DEGW,), jnp.float32)
        zbuf[i, :] = jnp.zeros((DEGW,), jnp.float32)
        return carry

    lax.fori_loop(0, CW, fill, 0)
    base = s * ROWS_PER_TILE
    for k in range(ROWS_PER_TILE // CW):
        pltpu.sync_copy(zbuf, deg_sh.at[pl.ds(base + k * CW, CW)])
    plsc.subcore_barrier()

    def body(j, carry):
        pltpu.sync_copy(ones_a, deg_sh.at[dst_v.at[j]], add=True)
        return carry

    lax.fori_loop(0, CH, body, 0)
    plsc.subcore_barrier()
    pltpu.sync_copy(deg_sh.at[pl.ds(base, ROWS_PER_TILE)],
                    deg_out.at[c, pl.ds(base, ROWS_PER_TILE)])


@functools.partial(
    pl.kernel,
    out_type=jax.ShapeDtypeStruct((NC, NP, C), jnp.float32),
    mesh=_mesh,
    scratch_types=[
        pltpu.VMEM((CPP, CW), jnp.int32),
        pltpu.VMEM((CPP, CW), jnp.int32),
        pltpu.VMEM((CW, C), jnp.float32),
        pltpu.VMEM((CW, C), jnp.float32),
        pltpu.VMEM_SHARED((NP, C), jnp.float32),
        pltpu.SemaphoreType.DMA,
        pltpu.SemaphoreType.DMA,
        pltpu.SemaphoreType.DMA,
        pltpu.SemaphoreType.DMA,
    ],
)
def _agg_kernel(ei_hbm, hp_hbm, acc_out,
                src_v, dst_v, buf0, buf1, acc_sh, g0, g1, s0, s1):
    c = lax.axis_index("c")
    s = lax.axis_index("s")

    def zf(i, carry):
        for k in range(C // 16):
            buf0[i, pl.ds(k * 16, 16)] = jnp.zeros((16,), jnp.float32)
        return carry

    lax.fori_loop(0, CW, zf, 0)
    base = s * ROWS_PER_TILE
    for k in range(ROWS_PER_TILE // CW):
        pltpu.sync_copy(buf0, acc_sh.at[pl.ds(base + k * CW, CW)])
    plsc.subcore_barrier()

    bufs = (buf0, buf1)
    gsems = (g0, g1)
    ssems = (s0, s1)

    def body(jj, carry):
        for bi in range(2):
            j = 2 * jj + bi
            buf, gs, ss = bufs[bi], gsems[bi], ssems[bi]
            pltpu.make_async_copy(hp_hbm.at[src_v.at[j]], buf, gs).wait()
            pltpu.async_copy(buf, acc_sh.at[dst_v.at[j]], ss, add=True).wait()
            jn = jnp.minimum(j + 2, CPP - 1)
            pltpu.async_copy(hp_hbm.at[src_v.at[jn]], buf, gs)
        return carry

    for p in range(NPASS):
        pltpu.sync_copy(ei_hbm.at[0, c, s, pl.ds(p * CPP, CPP)], src_v)
        pltpu.sync_copy(ei_hbm.at[1, c, s, pl.ds(p * CPP, CPP)], dst_v)
        pltpu.async_copy(hp_hbm.at[src_v.at[0]], buf0, g0)
        pltpu.async_copy(hp_hbm.at[src_v.at[1]], buf1, g1)
        lax.fori_loop(0, CPP // 2, body, 0)
        pltpu.make_async_copy(hp_hbm.at[src_v.at[CPP - 1]], buf0, g0).wait()
        pltpu.make_async_copy(hp_hbm.at[src_v.at[CPP - 1]], buf1, g1).wait()
    plsc.subcore_barrier()
    pltpu.sync_copy(acc_sh.at[pl.ds(base, ROWS_PER_TILE)],
                    acc_out.at[c, pl.ds(base, ROWS_PER_TILE)])


def _matmul_body(x_ref, w_ref, h_ref):
    h_ref[...] = jnp.dot(x_ref[...], w_ref[...],
                         preferred_element_type=jnp.float32)


_matmul = pl.pallas_call(
    _matmul_body,
    grid=(N // BR2,),
    in_specs=[
        pl.BlockSpec((BR2, C), lambda i: (i, 0)),
        pl.BlockSpec((C, C), lambda i: (0, 0)),
    ],
    out_specs=pl.BlockSpec((BR2, C), lambda i: (i, 0)),
    out_shape=jax.ShapeDtypeStruct((NP, C), jnp.float32),
)


def _scale_body(h_ref, deg_ref, hp_ref, dis_ref):
    dtot = deg_ref[0] + deg_ref[1] + 1.0
    dis = lax.rsqrt(dtot)[:, 0:1]
    hp_ref[...] = h_ref[...] * dis
    dis_ref[...] = dis


_scale = pl.pallas_call(
    _scale_body,
    grid=(NP // 1024,),
    in_specs=[
        pl.BlockSpec((1024, C), lambda i: (i, 0)),
        pl.BlockSpec((NC, 1024, DEGW), lambda i: (0, i, 0)),
    ],
    out_specs=[
        pl.BlockSpec((1024, C), lambda i: (i, 0)),
        pl.BlockSpec((1024, 1), lambda i: (i, 0)),
    ],
    out_shape=[
        jax.ShapeDtypeStruct((NP, C), jnp.float32),
        jax.ShapeDtypeStruct((NP, 1), jnp.float32),
    ],
)


def _finish_body(a_ref, hp_ref, dis_ref, b_ref, pw_ref, o_ref):
    t = (a_ref[0] + a_ref[1] + hp_ref[...]) * dis_ref[...] + b_ref[...]
    o_ref[...] = jnp.where(t >= 0.0, t, pw_ref[...] * t)


_finish = pl.pallas_call(
    _finish_body,
    grid=(N // BR2,),
    in_specs=[
        pl.BlockSpec((NC, BR2, C), lambda i: (0, i, 0)),
        pl.BlockSpec((BR2, C), lambda i: (i, 0)),
        pl.BlockSpec((BR2, 1), lambda i: (i, 0)),
        pl.BlockSpec((1, C), lambda i: (0, 0)),
        pl.BlockSpec((1, C), lambda i: (0, 0)),
    ],
    out_specs=pl.BlockSpec((BR2, C), lambda i: (i, 0)),
    out_shape=jax.ShapeDtypeStruct((N, C), jnp.float32),
)


_PAD2D = (np.arange(EP - E, dtype=np.int32) % (NP - N) + N).reshape(-1, CW)


def kernel(x, edge_index, W, b, prelu_w):
    ei = edge_index.astype(jnp.int32).reshape(2, E // CW, CW)
    pad3 = jnp.broadcast_to(_PAD2D, (2,) + _PAD2D.shape)
    eip = jnp.concatenate([ei, pad3], axis=1).reshape(2, NC, NS, CH, CW)

    deg = _deg_kernel(eip)
    h = _matmul(x, W)
    hp, dis = _scale(h, deg)
    acc = _agg_kernel(eip, hp)
    return _finish(acc, hp, dis, b.reshape(1, C), prelu_w.reshape(1, C))


# speedup vs baseline: 1.0278x; 1.0152x over previous
import functools

import numpy as np
import jax
import jax.numpy as jnp
from jax import lax
from jax.experimental import pallas as pl
from jax.experimental.pallas import tpu as pltpu
from jax.experimental.pallas import tpu_sc as plsc

N = 10000
C = 128
E = 320000
NC = 2
NS = 16
NW = NC * NS
NP = 10240
ROWS_PER_TILE = NP // NS
CW = 128
CH = 80
NPASS = 2
CPP = CH // NPASS
BR2 = 2000
EPT = CH * CW
EP = NW * EPT
DEGW = 16
BR = 640

_mesh = plsc.VectorSubcoreMesh(core_axis_name="c", subcore_axis_name="s")


@functools.partial(
    pl.kernel,
    out_type=jax.ShapeDtypeStruct((NC, NP, DEGW), jnp.float32),
    mesh=_mesh,
    scratch_types=[
        pltpu.VMEM((CH, CW), jnp.int32),
        pltpu.VMEM((CW, DEGW), jnp.float32),
        pltpu.VMEM((CW, DEGW), jnp.float32),
        pltpu.VMEM_SHARED((NP, DEGW), jnp.float32),
    ],
)
def _deg_kernel(ei_hbm, deg_out, dst_v, ones_a, zbuf, deg_sh):
    c = lax.axis_index("c")
    s = lax.axis_index("s")
    pltpu.sync_copy(ei_hbm.at[1, c, s], dst_v)

    def fill(i, carry):
        ones_a[i, :] = jnp.ones((DEGW,), jnp.float32)
        zbuf[i, :] = jnp.zeros((DEGW,), jnp.float32)
        return carry

    lax.fori_loop(0, CW, fill, 0)
    base = s * ROWS_PER_TILE
    for k in range(ROWS_PER_TILE // CW):
        pltpu.sync_copy(zbuf, deg_sh.at[pl.ds(base + k * CW, CW)])
    plsc.subcore_barrier()

    def body(j, carry):
        pltpu.sync_copy(ones_a, deg_sh.at[dst_v.at[j]], add=True)
        return carry

    lax.fori_loop(0, CH, body, 0)
    plsc.subcore_barrier()
    pltpu.sync_copy(deg_sh.at[pl.ds(base, ROWS_PER_TILE)],
                    deg_out.at[c, pl.ds(base, ROWS_PER_TILE)])


@functools.partial(
    pl.kernel,
    out_type=jax.ShapeDtypeStruct((NC, NP, C), jnp.float32),
    mesh=_mesh,
    scratch_types=[
        pltpu.VMEM((CPP, CW), jnp.int32),
        pltpu.VMEM((CPP, CW), jnp.int32),
        pltpu.VMEM((CW, C), jnp.float32),
        pltpu.VMEM((CW, C), jnp.float32),
        pltpu.VMEM_SHARED((NP, C), jnp.float32),
        pltpu.SemaphoreType.DMA,
        pltpu.SemaphoreType.DMA,
        pltpu.SemaphoreType.DMA,
        pltpu.SemaphoreType.DMA,
    ],
)
def _agg_kernel(ei_hbm, hp_hbm, acc_out,
                src_v, dst_v, buf0, buf1, acc_sh, g0, g1, s0, s1):
    c = lax.axis_index("c")
    s = lax.axis_index("s")

    def zf(i, carry):
        for k in range(C // 16):
            buf1[i, pl.ds(k * 16, 16)] = jnp.zeros((16,), jnp.float32)
        return carry

    pltpu.sync_copy(ei_hbm.at[0, c, s, pl.ds(0, CPP)], src_v)
    pltpu.sync_copy(ei_hbm.at[1, c, s, pl.ds(0, CPP)], dst_v)
    pltpu.async_copy(hp_hbm.at[src_v.at[0]], buf0, g0)

    lax.fori_loop(0, CW, zf, 0)
    base = s * ROWS_PER_TILE
    for k in range(ROWS_PER_TILE // CW):
        pltpu.sync_copy(buf1, acc_sh.at[pl.ds(base + k * CW, CW)])
    plsc.subcore_barrier()

    bufs = (buf0, buf1)
    gsems = (g0, g1)
    ssems = (s0, s1)

    def body(jj, carry):
        for bi in range(2):
            j = 2 * jj + bi
            buf, gs, ss = bufs[bi], gsems[bi], ssems[bi]
            pltpu.make_async_copy(hp_hbm.at[src_v.at[j]], buf, gs).wait()
            pltpu.async_copy(buf, acc_sh.at[dst_v.at[j]], ss, add=True).wait()
            jn = jnp.minimum(j + 2, CPP - 1)
            pltpu.async_copy(hp_hbm.at[src_v.at[jn]], buf, gs)
        return carry

    for p in range(NPASS):
        if p > 0:
            pltpu.sync_copy(ei_hbm.at[0, c, s, pl.ds(p * CPP, CPP)], src_v)
            pltpu.sync_copy(ei_hbm.at[1, c, s, pl.ds(p * CPP, CPP)], dst_v)
            pltpu.async_copy(hp_hbm.at[src_v.at[0]], buf0, g0)
        pltpu.async_copy(hp_hbm.at[src_v.at[1]], buf1, g1)
        lax.fori_loop(0, CPP // 2, body, 0)
        pltpu.make_async_copy(hp_hbm.at[src_v.at[CPP - 1]], buf0, g0).wait()
        pltpu.make_async_copy(hp_hbm.at[src_v.at[CPP - 1]], buf1, g1).wait()
    plsc.subcore_barrier()
    pltpu.sync_copy(acc_sh.at[pl.ds(base, ROWS_PER_TILE)],
                    acc_out.at[c, pl.ds(base, ROWS_PER_TILE)])


def _matmul_body(x_ref, w_ref, h_ref):
    h_ref[...] = jnp.dot(x_ref[...], w_ref[...],
                         preferred_element_type=jnp.float32)


_matmul = pl.pallas_call(
    _matmul_body,
    grid=(N // BR2,),
    in_specs=[
        pl.BlockSpec((BR2, C), lambda i: (i, 0)),
        pl.BlockSpec((C, C), lambda i: (0, 0)),
    ],
    out_specs=pl.BlockSpec((BR2, C), lambda i: (i, 0)),
    out_shape=jax.ShapeDtypeStruct((NP, C), jnp.float32),
)


def _scale_body(h_ref, deg_ref, hp_ref, dis_ref):
    dtot = deg_ref[0] + deg_ref[1] + 1.0
    dis = lax.rsqrt(dtot)[:, 0:1]
    hp_ref[...] = h_ref[...] * dis
    dis_ref[...] = dis


_scale = pl.pallas_call(
    _scale_body,
    grid=(NP // 2048,),
    in_specs=[
        pl.BlockSpec((2048, C), lambda i: (i, 0)),
        pl.BlockSpec((NC, 2048, DEGW), lambda i: (0, i, 0)),
    ],
    out_specs=[
        pl.BlockSpec((2048, C), lambda i: (i, 0)),
        pl.BlockSpec((2048, 1), lambda i: (i, 0)),
    ],
    out_shape=[
        jax.ShapeDtypeStruct((NP, C), jnp.float32),
        jax.ShapeDtypeStruct((NP, 1), jnp.float32),
    ],
)


def _finish_body(a_ref, hp_ref, dis_ref, b_ref, pw_ref, o_ref):
    t = (a_ref[0] + a_ref[1] + hp_ref[...]) * dis_ref[...] + b_ref[...]
    o_ref[...] = jnp.where(t >= 0.0, t, pw_ref[...] * t)


_finish = pl.pallas_call(
    _finish_body,
    grid=(N // BR2,),
    in_specs=[
        pl.BlockSpec((NC, BR2, C), lambda i: (0, i, 0)),
        pl.BlockSpec((BR2, C), lambda i: (i, 0)),
        pl.BlockSpec((BR2, 1), lambda i: (i, 0)),
        pl.BlockSpec((1, C), lambda i: (0, 0)),
        pl.BlockSpec((1, C), lambda i: (0, 0)),
    ],
    out_specs=pl.BlockSpec((BR2, C), lambda i: (i, 0)),
    out_shape=jax.ShapeDtypeStruct((N, C), jnp.float32),
)


_PAD2D = (np.arange(EP - E, dtype=np.int32) % (NP - N) + N).reshape(-1, CW)


def kernel(x, edge_index, W, b, prelu_w):
    ei = edge_index.astype(jnp.int32).reshape(2, E // CW, CW)
    pad3 = jnp.broadcast_to(_PAD2D, (2,) + _PAD2D.shape)
    eip = jnp.concatenate([ei, pad3], axis=1).reshape(2, NC, NS, CH, CW)

    deg = _deg_kernel(eip)
    h = _matmul(x, W)
    hp, dis = _scale(h, deg)
    acc = _agg_kernel(eip, hp)
    return _finish(acc, hp, dis, b.reshape(1, C), prelu_w.reshape(1, C))
